# baseline (speedup 1.0000x reference)
import functools

import jax
import jax.numpy as jnp
from jax import lax
from jax.experimental import pallas as pl
from jax.experimental.pallas import tpu as pltpu
from jax.experimental.pallas import tpu_sc as plsc

N = 256
E = 32
D = 128
H = 8
DH = D // H
SCALE = 1.0 / (DH ** 0.5)
NEG = -1e30

NW = 32
VPW = N // NW
NLANE = 16
NVEC = N // NLANE


def _dot(a, b):
    return jnp.dot(a, b, preferred_element_type=jnp.float32,
                   precision=lax.Precision.HIGHEST)


def _tc1_body(x_ref, ei_ref, ea_ref, wlin_t_ref, wedge_t_ref, wq_t_ref,
              wk_t_ref, wv_t_ref, bqkv_ref, a_out, c_out, v_out, has_out):
    Mf = (ei_ref[...] != 0).astype(jnp.float32)
    Mt = Mf.T
    Wh = _dot(x_ref[...], wlin_t_ref[...])
    We = _dot(ea_ref[...], wedge_t_ref[...])
    bq = bqkv_ref[0:1, :]
    bk = bqkv_ref[1:2, :]
    bv = bqkv_ref[2:3, :]
    Q = _dot(Wh, wq_t_ref[...]) + bq
    Kx = _dot(Wh, wk_t_ref[...])
    V = _dot(Wh, wv_t_ref[...]) + bv
    Ke = _dot(We, wk_t_ref[...]) + bk

    U = _dot(Mt, Mf)
    has_out[...] = (jnp.sum(Mt, axis=1, keepdims=True) > 0.0).astype(
        jnp.float32)

    for h in range(H):
        sl = slice(h * DH, (h + 1) * DH)
        osl = slice(h * N, (h + 1) * N)
        Qh, Kxh, Vh, Keh = Q[:, sl], Kx[:, sl], V[:, sl], Ke[:, sl]
        Ah = _dot(Qh, Kxh.T) * SCALE
        a_out[:, osl] = jnp.where(U > 0, Ah, NEG)
        Bh = _dot(Qh, Keh.T) * SCALE
        mB = jnp.max(jnp.where(Mt > 0, Bh, NEG), axis=1, keepdims=True)
        s = jnp.where(Mt > 0, jnp.exp(Bh - mB), 0.0)
        c_out[:, osl] = _dot(s, Mf)
    v_out[...] = V


def _sc_body(a_hbm, c_hbm, v_hbm, out_hbm, a_v, c_v, v_v, o_v):
    wid = lax.axis_index("s") * 2 + lax.axis_index("c")
    base = wid * VPW
    pltpu.sync_copy(a_hbm.at[pl.ds(base, VPW)], a_v)
    pltpu.sync_copy(c_hbm.at[pl.ds(base, VPW)], c_v)
    pltpu.sync_copy(v_hbm, v_v)
    lane_idx = [jnp.full((NLANE,), t, dtype=jnp.int32) for t in range(NLANE)]
    lanes = lax.iota(jnp.int32, NLANE)
    bfly = [lanes ^ k for k in (8, 4, 2, 1)]

    def _perm(vec, idx):
        return vec.at[idx].get(mode="promise_in_bounds")

    def _allmax(vec):
        for idx in bfly:
            vec = jnp.maximum(vec, _perm(vec, idx))
        return vec

    def _allsum(vec):
        for idx in bfly:
            vec = vec + _perm(vec, idx)
        return vec

    def h_loop(h, vi):
        col = h * N
        a = [a_v[vi, pl.ds(col + NLANE * j, NLANE)] for j in range(NVEC)]
        m = a[0]
        for j in range(1, NVEC):
            m = jnp.maximum(m, a[j])
        mA = _allmax(m)
        w = []
        z = None
        for j in range(NVEC):
            wj = c_v[vi, pl.ds(col + NLANE * j, NLANE)] * jnp.exp(a[j] - mA)
            w.append(wj)
            z = wj if z is None else z + wj
        rz = 1.0 / _allsum(z)
        drow = h * DH
        acc = None
        for j in range(NVEC):
            for t in range(NLANE):
                wb = w[j].at[lane_idx[t]].get(mode="promise_in_bounds")
                row = v_v[NLANE * j + t, pl.ds(drow, DH)]
                acc = wb * row if acc is None else acc + wb * row
        o_v[vi, pl.ds(drow, DH)] = acc * rz
        return vi

    def v_loop(vi, carry):
        lax.fori_loop(0, H, h_loop, vi)
        return carry

    lax.fori_loop(0, VPW, v_loop, 0)
    pltpu.sync_copy(o_v, out_hbm.at[pl.ds(base, VPW)])


def _tc2_body(attn_ref, has_ref, wout_t_ref, bout_ref, out_ref):
    o = _dot(attn_ref[...], wout_t_ref[...]) + bout_ref[...]
    o = jnp.where(has_ref[...] > 0.0, o, 0.0)
    out_ref[...] = jnp.maximum(o, 0.0)


@jax.jit
def kernel(x, edge_index, edge_attr, W_lin, W_edge, in_proj_w, in_proj_b,
           out_proj_w, out_proj_b):
    Wq, Wk, Wv = jnp.split(in_proj_w, 3, axis=0)
    bqkv = in_proj_b.reshape(3, D)

    tc1 = pl.pallas_call(
        _tc1_body,
        out_shape=(
            jax.ShapeDtypeStruct((N, H * N), jnp.float32),
            jax.ShapeDtypeStruct((N, H * N), jnp.float32),
            jax.ShapeDtypeStruct((N, D), jnp.float32),
            jax.ShapeDtypeStruct((N, 1), jnp.float32),
        ),
    )
    a_l, c_l, v_full, hasf = tc1(x, edge_index.astype(jnp.int32), edge_attr,
                                 W_lin.T, W_edge.T, Wq.T, Wk.T, Wv.T, bqkv)

    sc = functools.partial(
        pl.kernel,
        out_type=jax.ShapeDtypeStruct((N, D), jnp.float32),
        scratch_types=[
            pltpu.VMEM((VPW, H * N), jnp.float32),
            pltpu.VMEM((VPW, H * N), jnp.float32),
            pltpu.VMEM((N, D), jnp.float32),
            pltpu.VMEM((VPW, D), jnp.float32),
        ],
        mesh=plsc.VectorSubcoreMesh(core_axis_name="c", subcore_axis_name="s"),
    )(_sc_body)
    attn = sc(a_l, c_l, v_full)

    tc2 = pl.pallas_call(
        _tc2_body,
        out_shape=jax.ShapeDtypeStruct((N, D), jnp.float32),
    )
    return tc2(attn, hasf, out_proj_w.T, out_proj_b.reshape(1, D))


# speedup vs baseline: 1.1383x; 1.1383x over previous
import functools

import jax
import jax.numpy as jnp
from jax import lax
from jax.experimental import pallas as pl
from jax.experimental.pallas import tpu as pltpu
from jax.experimental.pallas import tpu_sc as plsc

N = 256
E = 32
D = 128
H = 8
DH = D // H
SCALE = 1.0 / (DH ** 0.5)
NEG = -1e30

NW = 32
VPW = N // NW
NLANE = 16
NVEC = N // NLANE


def _dot(a, b):
    return jnp.dot(a, b, preferred_element_type=jnp.float32,
                   precision=lax.Precision.HIGHEST)


def _tc1_body(x_ref, ei_ref, ea_ref, wlin_t_ref, wedge_t_ref, wq_t_ref,
              wk_t_ref, wv_t_ref, bqkv_ref, a_out, c_out, v_out, has_out):
    Mf = (ei_ref[...] != 0).astype(jnp.float32)
    Mt = Mf.T
    Wh = _dot(x_ref[...], wlin_t_ref[...])
    We = _dot(ea_ref[...], wedge_t_ref[...])
    bq = bqkv_ref[0:1, :]
    bk = bqkv_ref[1:2, :]
    bv = bqkv_ref[2:3, :]
    Q = _dot(Wh, wq_t_ref[...]) + bq
    Kx = _dot(Wh, wk_t_ref[...])
    V = _dot(Wh, wv_t_ref[...]) + bv
    Ke = _dot(We, wk_t_ref[...]) + bk

    U = _dot(Mt, Mf)
    has_out[...] = (jnp.sum(Mt, axis=1, keepdims=True) > 0.0).astype(
        jnp.float32)

    for h in range(H):
        sl = slice(h * DH, (h + 1) * DH)
        osl = slice(h * N, (h + 1) * N)
        Qh, Kxh, Vh, Keh = Q[:, sl], Kx[:, sl], V[:, sl], Ke[:, sl]
        Ah = _dot(Qh, Kxh.T) * SCALE
        a_out[:, osl] = jnp.where(U > 0, Ah, NEG)
        Bh = _dot(Qh, Keh.T) * SCALE
        mB = jnp.max(jnp.where(Mt > 0, Bh, NEG), axis=1, keepdims=True)
        s = jnp.where(Mt > 0, jnp.exp(Bh - mB), 0.0)
        c_out[:, osl] = _dot(s, Mf)
    v_out[...] = V


def _sc_body(a_hbm, c_hbm, v_hbm, out_hbm, a_v, c_v, v_v, o_v):
    wid = lax.axis_index("s") * 2 + lax.axis_index("c")
    base = wid * VPW
    pltpu.sync_copy(a_hbm.at[pl.ds(base, VPW)], a_v)
    pltpu.sync_copy(c_hbm.at[pl.ds(base, VPW)], c_v)
    pltpu.sync_copy(v_hbm, v_v)
    lane_idx = [jnp.full((NLANE,), t, dtype=jnp.int32) for t in range(NLANE)]
    lanes = lax.iota(jnp.int32, NLANE)
    bfly = [lanes ^ k for k in (8, 4, 2, 1)]

    def _perm(vec, idx):
        return vec.at[idx].get(mode="promise_in_bounds")

    def _allmax(vec):
        for idx in bfly:
            vec = jnp.maximum(vec, _perm(vec, idx))
        return vec

    def _allsum(vec):
        for idx in bfly:
            vec = vec + _perm(vec, idx)
        return vec

    def _tree(vals, op):
        while len(vals) > 1:
            vals = [op(vals[i], vals[i + 1]) if i + 1 < len(vals) else vals[i]
                    for i in range(0, len(vals), 2)]
        return vals[0]

    def h_loop(h, vi):
        col = h * N
        a = [a_v[vi, pl.ds(col + NLANE * j, NLANE)] for j in range(NVEC)]
        mA = _allmax(_tree(a, jnp.maximum))
        w = [c_v[vi, pl.ds(col + NLANE * j, NLANE)] * jnp.exp(a[j] - mA)
             for j in range(NVEC)]
        rz = 1.0 / _allsum(_tree(list(w), jnp.add))
        drow = h * DH
        accs = [None] * 8
        for j in range(NVEC):
            for t in range(NLANE):
                wb = w[j].at[lane_idx[t]].get(mode="promise_in_bounds")
                row = v_v[NLANE * j + t, pl.ds(drow, DH)]
                k = t % 8
                accs[k] = wb * row if accs[k] is None else accs[k] + wb * row
        o_v[vi, pl.ds(drow, DH)] = _tree(accs, jnp.add) * rz
        return vi

    def v_loop(vi, carry):
        lax.fori_loop(0, H, h_loop, vi)
        return carry

    lax.fori_loop(0, VPW, v_loop, 0)
    pltpu.sync_copy(o_v, out_hbm.at[pl.ds(base, VPW)])


def _tc2_body(attn_ref, has_ref, wout_t_ref, bout_ref, out_ref):
    o = _dot(attn_ref[...], wout_t_ref[...]) + bout_ref[...]
    o = jnp.where(has_ref[...] > 0.0, o, 0.0)
    out_ref[...] = jnp.maximum(o, 0.0)


@jax.jit
def kernel(x, edge_index, edge_attr, W_lin, W_edge, in_proj_w, in_proj_b,
           out_proj_w, out_proj_b):
    Wq, Wk, Wv = jnp.split(in_proj_w, 3, axis=0)
    bqkv = in_proj_b.reshape(3, D)

    tc1 = pl.pallas_call(
        _tc1_body,
        out_shape=(
            jax.ShapeDtypeStruct((N, H * N), jnp.float32),
            jax.ShapeDtypeStruct((N, H * N), jnp.float32),
            jax.ShapeDtypeStruct((N, D), jnp.float32),
            jax.ShapeDtypeStruct((N, 1), jnp.float32),
        ),
    )
    a_l, c_l, v_full, hasf = tc1(x, edge_index.astype(jnp.int32), edge_attr,
                                 W_lin.T, W_edge.T, Wq.T, Wk.T, Wv.T, bqkv)

    sc = functools.partial(
        pl.kernel,
        out_type=jax.ShapeDtypeStruct((N, D), jnp.float32),
        scratch_types=[
            pltpu.VMEM((VPW, H * N), jnp.float32),
            pltpu.VMEM((VPW, H * N), jnp.float32),
            pltpu.VMEM((N, D), jnp.float32),
            pltpu.VMEM((VPW, D), jnp.float32),
        ],
        mesh=plsc.VectorSubcoreMesh(core_axis_name="c", subcore_axis_name="s"),
    )(_sc_body)
    attn = sc(a_l, c_l, v_full)

    tc2 = pl.pallas_call(
        _tc2_body,
        out_shape=jax.ShapeDtypeStruct((N, D), jnp.float32),
    )
    return tc2(attn, hasf, out_proj_w.T, out_proj_b.reshape(1, D))
